# baseline (speedup 1.0000x reference)
import functools

import jax
import jax.numpy as jnp
from jax import lax
from jax.experimental import pallas as pl
from jax.experimental.pallas import tpu as pltpu
from jax.experimental.pallas import tpu_sc as plsc

N_NODES = 10000
N_EDGES = 320000
F_IN = 128
H = 64
D = 2 * H

NC = 2
NS = 16
NW = NC * NS
E_PER_W = N_EDGES // NW
CHUNK = 80
NCHUNK = E_PER_W // CHUNK
ROWS_PER_TILE = 624
ROWS_LAST = N_NODES - ROWS_PER_TILE * (NS - 1)
EPS = 1e-7

RBLK = 2000
NBLK = N_NODES // RBLK


def _layer_norm(h, g, b):
    mu = jnp.mean(h, axis=-1, keepdims=True)
    var = jnp.mean((h - mu) ** 2, axis=-1, keepdims=True)
    return (h - mu) * lax.rsqrt(var + 1e-5) * g + b


def _row_spec(shape):
    return pl.BlockSpec(shape, lambda i: (i,) + (0,) * (len(shape) - 1))


def _full_spec(shape):
    return pl.BlockSpec(shape, lambda i: (0,) * len(shape))


def _dense_a_body(x_ref, we_ref, be_ref, t_ref, y_ref, aux_ref):
    y = jnp.dot(x_ref[...], we_ref[...], preferred_element_type=jnp.float32)
    y = y + be_ref[...]
    y_ref[...] = y
    m = jax.nn.relu(y) + EPS
    t = t_ref[0, 0]
    mf = jnp.max(m * t, axis=0, keepdims=True)
    row = jnp.concatenate([mf, jnp.full((1, H), t, jnp.float32)], axis=1)
    aux_ref[...] = jnp.broadcast_to(row, (8, D))


def _dense_a(x, W_enc, b_enc, t):
    return pl.pallas_call(
        _dense_a_body,
        out_shape=(
            jax.ShapeDtypeStruct((N_NODES, H), jnp.float32),
            jax.ShapeDtypeStruct((8, D), jnp.float32),
        ),
    )(x, W_enc, b_enc.reshape(1, H), t.reshape(1, 1))


def _aggregate(nd_ref, x):
    nd = nd_ref[0] + nd_ref[1]
    den = nd[:, :H]
    num = nd[:, H:]
    agg = num / jnp.where(den > 0.0, den, 1.0)
    return agg + x


def _mlp(h, W1_ref, b1_ref, gm_ref, bm_ref, W2_ref, b2_ref):
    h = jnp.dot(h, W1_ref[...], preferred_element_type=jnp.float32) + b1_ref[...]
    h = _layer_norm(h, gm_ref[...], bm_ref[...])
    h = jax.nn.relu(h)
    return jnp.dot(h, W2_ref[...], preferred_element_type=jnp.float32) + b2_ref[...]


def _dense_b_body(nd_ref, y_ref, t_ref, W1_ref, b1_ref, gm_ref, bm_ref,
                  W2_ref, b2_ref, z_ref, aux_ref):
    out = _aggregate(nd_ref, y_ref[...])
    z = _mlp(out, W1_ref, b1_ref, gm_ref, bm_ref, W2_ref, b2_ref)
    z_ref[...] = z
    m = jax.nn.relu(z) + EPS
    t = t_ref[0, 0]
    mf = jnp.max(m * t, axis=0, keepdims=True)
    row = jnp.concatenate([mf, jnp.full((1, H), t, jnp.float32)], axis=1)
    aux_ref[...] = jnp.broadcast_to(row, (8, D))


def _dense_b(nd, y, t, W1, b1, g_m, b_m, W2, b2):
    return pl.pallas_call(
        _dense_b_body,
        out_shape=(
            jax.ShapeDtypeStruct((N_NODES, H), jnp.float32),
            jax.ShapeDtypeStruct((8, D), jnp.float32),
        ),
    )(nd, y, t.reshape(1, 1), W1, b1.reshape(1, D), g_m.reshape(1, D),
      b_m.reshape(1, D), W2, b2.reshape(1, H))


def _dense_c_body(nd_ref, z_ref, W1_ref, b1_ref, gm_ref, bm_ref, W2_ref,
                  b2_ref, gl_ref, bl_ref, gn_ref, bn_ref, wo_ref, bo_ref,
                  o_ref):
    out = _aggregate(nd_ref, z_ref[...])
    z2 = _mlp(out, W1_ref, b1_ref, gm_ref, bm_ref, W2_ref, b2_ref)
    h = jax.nn.relu(_layer_norm(z2, gl_ref[...], bl_ref[...]))
    cat = jnp.concatenate([z_ref[...], h], axis=1)
    cat = jax.nn.relu(_layer_norm(cat, gn_ref[...], bn_ref[...]))
    o_ref[...] = (jnp.dot(cat, wo_ref[...], preferred_element_type=jnp.float32)
                  + bo_ref[...])


def _dense_c(nd, z, W1, b1, g_m, b_m, W2, b2, g_ln1, b_ln1, g_norm, b_norm,
             W_out, b_out):
    return pl.pallas_call(
        _dense_c_body,
        grid=(NBLK,),
        in_specs=[
            pl.BlockSpec((2, RBLK, D), lambda i: (0, i, 0)),
            _row_spec((RBLK, H)),
            _full_spec((H, D)),
            _full_spec((1, D)),
            _full_spec((1, D)),
            _full_spec((1, D)),
            _full_spec((D, H)),
            _full_spec((1, H)),
            _full_spec((1, H)),
            _full_spec((1, H)),
            _full_spec((1, F_IN)),
            _full_spec((1, F_IN)),
            _full_spec((F_IN, 1)),
            _full_spec((1, 1)),
        ],
        out_specs=_row_spec((RBLK, 1)),
        out_shape=jax.ShapeDtypeStruct((N_NODES, 1), jnp.float32),
    )(nd, z, W1, b1.reshape(1, D), g_m.reshape(1, D), b_m.reshape(1, D),
      W2, b2.reshape(1, H), g_ln1.reshape(1, H), b_ln1.reshape(1, H),
      g_norm.reshape(1, F_IN), b_norm.reshape(1, F_IN), W_out,
      b_out.reshape(1, 1))


UNROLL = 4


def _sc_body(y_hbm, eidx_hbm, aux_hbm, zeros_hbm, out_hbm,
             idx_a, idx_b, sidx_a, sidx_b, in_a, in_b, st_a, st_b, aux_v, acc,
             isem_a, isem_b, gsem_a, gsem_b, ssem_a, ssem_b):
    c = lax.axis_index("c")
    s = lax.axis_index("s")
    wid = s * NC + c

    pltpu.sync_copy(aux_hbm, aux_v)

    row0 = s * ROWS_PER_TILE

    @pl.when(s < NS - 1)
    def _():
        pltpu.sync_copy(zeros_hbm.at[pl.ds(row0, ROWS_PER_TILE)],
                        acc.at[pl.ds(row0, ROWS_PER_TILE)])

    @pl.when(s == NS - 1)
    def _():
        pltpu.sync_copy(zeros_hbm.at[pl.ds(row0, ROWS_LAST)],
                        acc.at[pl.ds(row0, ROWS_LAST)])

    plsc.subcore_barrier()

    mfs = [aux_v[0, pl.ds(16 * g, 16)] for g in range(H // 16)]
    tv = aux_v[0, pl.ds(H, 16)]

    edges = eidx_hbm.at[wid]

    def fetch_idx(j, idxbuf, sem):
        return pltpu.async_copy(edges.at[j], idxbuf, sem)

    def gather(idxbuf, inbuf, sem):
        return pltpu.async_copy(y_hbm.at[idxbuf.at[0]], inbuf, sem)

    def wait_gather(idxbuf, inbuf, sem):
        pltpu.make_async_copy(y_hbm.at[idxbuf.at[0]], inbuf, sem).wait()

    def scatter(stbuf, sidx, sem):
        return pltpu.async_copy(stbuf, acc.at[sidx.at[0]], sem, add=True)

    def wait_scatter(stbuf, sidx, sem):
        pltpu.make_async_copy(stbuf, acc.at[sidx.at[0]], sem).wait()

    def copy_dst_idx(idxbuf, sidx):
        for k in range(CHUNK // 16):
            sidx[0, pl.ds(16 * k, 16)] = idxbuf[1, pl.ds(16 * k, 16)]

    def compute(inbuf, stbuf):
        NG = H // 16

        def rows(r, carry):
            base = r * UNROLL
            idxs = [(k, g) for k in range(UNROLL) for g in range(NG)]
            ys = [inbuf[base + k, pl.ds(16 * g, 16)] for k, g in idxs]
            ms = [jnp.maximum(yv, 0.0) + EPS for yv in ys]
            ps = [jnp.exp(ms[i] * tv - mfs[g]) for i, (k, g) in enumerate(idxs)]
            qs = [p * m for p, m in zip(ps, ms)]
            for i, (k, g) in enumerate(idxs):
                stbuf[base + k, pl.ds(16 * g, 16)] = ps[i]
            for i, (k, g) in enumerate(idxs):
                stbuf[base + k, pl.ds(H + 16 * g, 16)] = qs[i]
            return carry

        lax.fori_loop(0, CHUNK // UNROLL, rows, 0)

    def phase(i, j, idxbuf, sidx, inbuf, stbuf, isem, gsem, ssem):
        wait_gather(idxbuf, inbuf, gsem)

        @pl.when(i > 0)
        def _():
            wait_scatter(stbuf, sidx, ssem)

        copy_dst_idx(idxbuf, sidx)

        @pl.when(j + 2 < NCHUNK)
        def _():
            fetch_idx(j + 2, idxbuf, isem)

        compute(inbuf, stbuf)
        scatter(stbuf, sidx, ssem)

        @pl.when(j + 2 < NCHUNK)
        def _():
            pltpu.make_async_copy(edges.at[j + 2], idxbuf, isem).wait()
            gather(idxbuf, inbuf, gsem)

    fetch_idx(0, idx_a, isem_a).wait()
    fetch_idx(1, idx_b, isem_b).wait()
    gather(idx_a, in_a, gsem_a)
    gather(idx_b, in_b, gsem_b)

    def pair(i, carry):
        j0 = 2 * i
        phase(i, j0, idx_a, sidx_a, in_a, st_a, isem_a, gsem_a, ssem_a)
        phase(i, j0 + 1, idx_b, sidx_b, in_b, st_b, isem_b, gsem_b, ssem_b)
        return carry

    lax.fori_loop(0, NCHUNK // 2, pair, 0)

    if NCHUNK % 2:
        wait_gather(idx_a, in_a, gsem_a)
        wait_scatter(st_a, sidx_a, ssem_a)
        copy_dst_idx(idx_a, sidx_a)
        compute(in_a, st_a)
        scatter(st_a, sidx_a, ssem_a)

    wait_scatter(st_a, sidx_a, ssem_a)
    wait_scatter(st_b, sidx_b, ssem_b)
    plsc.subcore_barrier()

    @pl.when(s < NS - 1)
    def _():
        pltpu.sync_copy(acc.at[pl.ds(row0, ROWS_PER_TILE)],
                        out_hbm.at[c].at[pl.ds(row0, ROWS_PER_TILE)])

    @pl.when(s == NS - 1)
    def _():
        pltpu.sync_copy(acc.at[pl.ds(row0, ROWS_LAST)],
                        out_hbm.at[c].at[pl.ds(row0, ROWS_LAST)])


@functools.cache
def _make_sc_scatter():
    return pl.kernel(
        _sc_body,
        out_type=jax.ShapeDtypeStruct((NC, N_NODES, D), jnp.float32),
        mesh=plsc.VectorSubcoreMesh(core_axis_name="c", subcore_axis_name="s",
                                    num_cores=NC, num_subcores=NS),
        compiler_params=pltpu.CompilerParams(use_tc_tiling_on_sc=False),
        scratch_types=[
            pltpu.VMEM((2, CHUNK), jnp.int32),
            pltpu.VMEM((2, CHUNK), jnp.int32),
            pltpu.VMEM((1, CHUNK), jnp.int32),
            pltpu.VMEM((1, CHUNK), jnp.int32),
            pltpu.VMEM((CHUNK, H), jnp.float32),
            pltpu.VMEM((CHUNK, H), jnp.float32),
            pltpu.VMEM((CHUNK, D), jnp.float32),
            pltpu.VMEM((CHUNK, D), jnp.float32),
            pltpu.VMEM((8, D), jnp.float32),
            pltpu.VMEM_SHARED((N_NODES, D), jnp.float32),
            pltpu.SemaphoreType.DMA,
            pltpu.SemaphoreType.DMA,
            pltpu.SemaphoreType.DMA,
            pltpu.SemaphoreType.DMA,
            pltpu.SemaphoreType.DMA,
            pltpu.SemaphoreType.DMA,
        ],
    )


def _sc_scatter(y, eidx, aux, zeros):
    return _make_sc_scatter()(y, eidx, aux, zeros)


def kernel(x, edge_index, W_enc, b_enc, t, W1, b1, g_m, b_m, W2, b2,
           g_ln1, b_ln1, g_norm, b_norm, W_out, b_out):
    eidx = edge_index.reshape(2, NW, NCHUNK, CHUNK).transpose(1, 2, 0, 3)
    zeros = jnp.zeros((N_NODES, D), jnp.float32)

    y, aux1 = _dense_a(x, W_enc, b_enc, t)
    nd1 = _sc_scatter(y, eidx, aux1, zeros)
    z, aux2 = _dense_b(nd1, y, t, W1, b1, g_m, b_m, W2, b2)
    nd2 = _sc_scatter(z, eidx, aux2, zeros)
    return _dense_c(nd2, z, W1, b1, g_m, b_m, W2, b2, g_ln1, b_ln1,
                    g_norm, b_norm, W_out, b_out)


# speedup vs baseline: 1.0362x; 1.0362x over previous
import functools

import jax
import jax.numpy as jnp
from jax import lax
from jax.experimental import pallas as pl
from jax.experimental.pallas import tpu as pltpu
from jax.experimental.pallas import tpu_sc as plsc

N_NODES = 10000
N_EDGES = 320000
F_IN = 128
H = 64
D = 2 * H

NC = 2
NS = 16
NW = NC * NS
E_PER_W = N_EDGES // NW
CHUNK = 80
NCHUNK = E_PER_W // CHUNK
ROWS_PER_TILE = 624
ROWS_LAST = N_NODES - ROWS_PER_TILE * (NS - 1)
EPS = 1e-7

RBLK = 2000
NBLK = N_NODES // RBLK


def _layer_norm(h, g, b):
    mu = jnp.mean(h, axis=-1, keepdims=True)
    var = jnp.mean((h - mu) ** 2, axis=-1, keepdims=True)
    return (h - mu) * lax.rsqrt(var + 1e-5) * g + b


def _row_spec(shape):
    return pl.BlockSpec(shape, lambda i: (i,) + (0,) * (len(shape) - 1))


def _full_spec(shape):
    return pl.BlockSpec(shape, lambda i: (0,) * len(shape))


def _dense_a_body(x_ref, we_ref, be_ref, t_ref, y_ref, aux_ref):
    y = jnp.dot(x_ref[...], we_ref[...], preferred_element_type=jnp.float32)
    y = y + be_ref[...]
    y_ref[...] = y
    m = jax.nn.relu(y) + EPS
    t = t_ref[0, 0]
    mf = jnp.max(m * t, axis=0, keepdims=True)
    row = jnp.concatenate([mf, jnp.full((1, H), t, jnp.float32)], axis=1)
    aux_ref[...] = jnp.broadcast_to(row, (8, D))


def _dense_a(x, W_enc, b_enc, t):
    return pl.pallas_call(
        _dense_a_body,
        out_shape=(
            jax.ShapeDtypeStruct((N_NODES, H), jnp.float32),
            jax.ShapeDtypeStruct((8, D), jnp.float32),
        ),
    )(x, W_enc, b_enc.reshape(1, H), t.reshape(1, 1))


def _aggregate(nd_ref, x):
    nd = nd_ref[0] + nd_ref[1]
    den = nd[:, :H]
    num = nd[:, H:]
    agg = num / jnp.where(den > 0.0, den, 1.0)
    return agg + x


def _mlp(h, W1_ref, b1_ref, gm_ref, bm_ref, W2_ref, b2_ref):
    h = jnp.dot(h, W1_ref[...], preferred_element_type=jnp.float32) + b1_ref[...]
    h = _layer_norm(h, gm_ref[...], bm_ref[...])
    h = jax.nn.relu(h)
    return jnp.dot(h, W2_ref[...], preferred_element_type=jnp.float32) + b2_ref[...]


def _dense_b_body(nd_ref, y_ref, t_ref, W1_ref, b1_ref, gm_ref, bm_ref,
                  W2_ref, b2_ref, z_ref, aux_ref):
    out = _aggregate(nd_ref, y_ref[...])
    z = _mlp(out, W1_ref, b1_ref, gm_ref, bm_ref, W2_ref, b2_ref)
    z_ref[...] = z
    m = jax.nn.relu(z) + EPS
    t = t_ref[0, 0]
    mf = jnp.max(m * t, axis=0, keepdims=True)
    row = jnp.concatenate([mf, jnp.full((1, H), t, jnp.float32)], axis=1)
    aux_ref[...] = jnp.broadcast_to(row, (8, D))


def _dense_b(nd, y, t, W1, b1, g_m, b_m, W2, b2):
    return pl.pallas_call(
        _dense_b_body,
        out_shape=(
            jax.ShapeDtypeStruct((N_NODES, H), jnp.float32),
            jax.ShapeDtypeStruct((8, D), jnp.float32),
        ),
    )(nd, y, t.reshape(1, 1), W1, b1.reshape(1, D), g_m.reshape(1, D),
      b_m.reshape(1, D), W2, b2.reshape(1, H))


def _dense_c_body(nd_ref, z_ref, W1_ref, b1_ref, gm_ref, bm_ref, W2_ref,
                  b2_ref, gl_ref, bl_ref, gn_ref, bn_ref, wo_ref, bo_ref,
                  o_ref):
    out = _aggregate(nd_ref, z_ref[...])
    z2 = _mlp(out, W1_ref, b1_ref, gm_ref, bm_ref, W2_ref, b2_ref)
    h = jax.nn.relu(_layer_norm(z2, gl_ref[...], bl_ref[...]))
    cat = jnp.concatenate([z_ref[...], h], axis=1)
    cat = jax.nn.relu(_layer_norm(cat, gn_ref[...], bn_ref[...]))
    o_ref[...] = (jnp.dot(cat, wo_ref[...], preferred_element_type=jnp.float32)
                  + bo_ref[...])


def _dense_c(nd, z, W1, b1, g_m, b_m, W2, b2, g_ln1, b_ln1, g_norm, b_norm,
             W_out, b_out):
    return pl.pallas_call(
        _dense_c_body,
        grid=(NBLK,),
        in_specs=[
            pl.BlockSpec((2, RBLK, D), lambda i: (0, i, 0)),
            _row_spec((RBLK, H)),
            _full_spec((H, D)),
            _full_spec((1, D)),
            _full_spec((1, D)),
            _full_spec((1, D)),
            _full_spec((D, H)),
            _full_spec((1, H)),
            _full_spec((1, H)),
            _full_spec((1, H)),
            _full_spec((1, F_IN)),
            _full_spec((1, F_IN)),
            _full_spec((F_IN, 1)),
            _full_spec((1, 1)),
        ],
        out_specs=_row_spec((RBLK, 1)),
        out_shape=jax.ShapeDtypeStruct((N_NODES, 1), jnp.float32),
    )(nd, z, W1, b1.reshape(1, D), g_m.reshape(1, D), b_m.reshape(1, D),
      W2, b2.reshape(1, H), g_ln1.reshape(1, H), b_ln1.reshape(1, H),
      g_norm.reshape(1, F_IN), b_norm.reshape(1, F_IN), W_out,
      b_out.reshape(1, 1))


UNROLL = 4


def _sc_body(y_hbm, eidx_hbm, aux_hbm, zeros_hbm, out_hbm,
             idx_a, idx_b, idx_c, sidx_a, sidx_b, sidx_c,
             in_a, in_b, in_c, st_a, st_b, st_c, aux_v, acc,
             isem_a, isem_b, isem_c, gsem_a, gsem_b, gsem_c,
             ssem_a, ssem_b, ssem_c):
    c = lax.axis_index("c")
    s = lax.axis_index("s")
    wid = s * NC + c

    pltpu.sync_copy(aux_hbm, aux_v)

    row0 = s * ROWS_PER_TILE

    @pl.when(s < NS - 1)
    def _():
        pltpu.sync_copy(zeros_hbm.at[pl.ds(row0, ROWS_PER_TILE)],
                        acc.at[pl.ds(row0, ROWS_PER_TILE)])

    @pl.when(s == NS - 1)
    def _():
        pltpu.sync_copy(zeros_hbm.at[pl.ds(row0, ROWS_LAST)],
                        acc.at[pl.ds(row0, ROWS_LAST)])

    plsc.subcore_barrier()

    mfs = [aux_v[0, pl.ds(16 * g, 16)] for g in range(H // 16)]
    tv = aux_v[0, pl.ds(H, 16)]

    edges = eidx_hbm.at[wid]

    def fetch_idx(j, idxbuf, sem):
        return pltpu.async_copy(edges.at[j], idxbuf, sem)

    def gather(idxbuf, inbuf, sem):
        return pltpu.async_copy(y_hbm.at[idxbuf.at[0]], inbuf, sem)

    def wait_gather(idxbuf, inbuf, sem):
        pltpu.make_async_copy(y_hbm.at[idxbuf.at[0]], inbuf, sem).wait()

    def scatter(stbuf, sidx, sem):
        return pltpu.async_copy(stbuf, acc.at[sidx.at[0]], sem, add=True)

    def wait_scatter(stbuf, sidx, sem):
        pltpu.make_async_copy(stbuf, acc.at[sidx.at[0]], sem).wait()

    def copy_dst_idx(idxbuf, sidx):
        for k in range(CHUNK // 16):
            sidx[0, pl.ds(16 * k, 16)] = idxbuf[1, pl.ds(16 * k, 16)]

    def compute(inbuf, stbuf):
        NG = H // 16

        def rows(r, carry):
            base = r * UNROLL
            idxs = [(k, g) for k in range(UNROLL) for g in range(NG)]
            ys = [inbuf[base + k, pl.ds(16 * g, 16)] for k, g in idxs]
            ms = [jnp.maximum(yv, 0.0) + EPS for yv in ys]
            ps = [jnp.exp(ms[i] * tv - mfs[g]) for i, (k, g) in enumerate(idxs)]
            qs = [p * m for p, m in zip(ps, ms)]
            for i, (k, g) in enumerate(idxs):
                stbuf[base + k, pl.ds(16 * g, 16)] = ps[i]
            for i, (k, g) in enumerate(idxs):
                stbuf[base + k, pl.ds(H + 16 * g, 16)] = qs[i]
            return carry

        lax.fori_loop(0, CHUNK // UNROLL, rows, 0)

    def phase(i, j, idxbuf, sidx, inbuf, stbuf, isem, gsem, ssem):
        wait_gather(idxbuf, inbuf, gsem)

        @pl.when(i > 0)
        def _():
            wait_scatter(stbuf, sidx, ssem)

        copy_dst_idx(idxbuf, sidx)

        @pl.when(j + 3 < NCHUNK)
        def _():
            fetch_idx(j + 3, idxbuf, isem)

        compute(inbuf, stbuf)
        scatter(stbuf, sidx, ssem)

        @pl.when(j + 3 < NCHUNK)
        def _():
            pltpu.make_async_copy(edges.at[j + 3], idxbuf, isem).wait()
            gather(idxbuf, inbuf, gsem)

    fetch_idx(0, idx_a, isem_a).wait()
    fetch_idx(1, idx_b, isem_b).wait()
    fetch_idx(2, idx_c, isem_c).wait()
    gather(idx_a, in_a, gsem_a)
    gather(idx_b, in_b, gsem_b)
    gather(idx_c, in_c, gsem_c)

    def triple(i, carry):
        j0 = 3 * i
        phase(i, j0, idx_a, sidx_a, in_a, st_a, isem_a, gsem_a, ssem_a)
        phase(i, j0 + 1, idx_b, sidx_b, in_b, st_b, isem_b, gsem_b, ssem_b)
        phase(i, j0 + 2, idx_c, sidx_c, in_c, st_c, isem_c, gsem_c, ssem_c)
        return carry

    lax.fori_loop(0, NCHUNK // 3, triple, 0)

    _tail = NCHUNK - 3 * (NCHUNK // 3)
    _sets = [(idx_a, sidx_a, in_a, st_a, isem_a, gsem_a, ssem_a),
             (idx_b, sidx_b, in_b, st_b, isem_b, gsem_b, ssem_b),
             (idx_c, sidx_c, in_c, st_c, isem_c, gsem_c, ssem_c)]
    for k in range(_tail):
        phase(1, NCHUNK - _tail + k, *_sets[k])

    wait_scatter(st_a, sidx_a, ssem_a)
    wait_scatter(st_b, sidx_b, ssem_b)
    wait_scatter(st_c, sidx_c, ssem_c)
    plsc.subcore_barrier()

    @pl.when(s < NS - 1)
    def _():
        pltpu.sync_copy(acc.at[pl.ds(row0, ROWS_PER_TILE)],
                        out_hbm.at[c].at[pl.ds(row0, ROWS_PER_TILE)])

    @pl.when(s == NS - 1)
    def _():
        pltpu.sync_copy(acc.at[pl.ds(row0, ROWS_LAST)],
                        out_hbm.at[c].at[pl.ds(row0, ROWS_LAST)])


@functools.cache
def _make_sc_scatter():
    return pl.kernel(
        _sc_body,
        out_type=jax.ShapeDtypeStruct((NC, N_NODES, D), jnp.float32),
        mesh=plsc.VectorSubcoreMesh(core_axis_name="c", subcore_axis_name="s",
                                    num_cores=NC, num_subcores=NS),
        compiler_params=pltpu.CompilerParams(use_tc_tiling_on_sc=False),
        scratch_types=(
            [pltpu.VMEM((2, CHUNK), jnp.int32)] * 3
            + [pltpu.VMEM((1, CHUNK), jnp.int32)] * 3
            + [pltpu.VMEM((CHUNK, H), jnp.float32)] * 3
            + [pltpu.VMEM((CHUNK, D), jnp.float32)] * 3
            + [pltpu.VMEM((8, D), jnp.float32)]
            + [pltpu.VMEM_SHARED((N_NODES, D), jnp.float32)]
            + [pltpu.SemaphoreType.DMA] * 9
        ),
    )


def _sc_scatter(y, eidx, aux, zeros):
    return _make_sc_scatter()(y, eidx, aux, zeros)


def kernel(x, edge_index, W_enc, b_enc, t, W1, b1, g_m, b_m, W2, b2,
           g_ln1, b_ln1, g_norm, b_norm, W_out, b_out):
    eidx = edge_index.reshape(2, NW, NCHUNK, CHUNK).transpose(1, 2, 0, 3)
    zeros = jnp.zeros((N_NODES, D), jnp.float32)

    y, aux1 = _dense_a(x, W_enc, b_enc, t)
    nd1 = _sc_scatter(y, eidx, aux1, zeros)
    z, aux2 = _dense_b(nd1, y, t, W1, b1, g_m, b_m, W2, b2)
    nd2 = _sc_scatter(z, eidx, aux2, zeros)
    return _dense_c(nd2, z, W1, b1, g_m, b_m, W2, b2, g_ln1, b_ln1,
                    g_norm, b_norm, W_out, b_out)


# speedup vs baseline: 1.1130x; 1.0741x over previous
import functools

import jax
import jax.numpy as jnp
from jax import lax
from jax.experimental import pallas as pl
from jax.experimental.pallas import tpu as pltpu
from jax.experimental.pallas import tpu_sc as plsc

N_NODES = 10000
N_EDGES = 320000
F_IN = 128
H = 64
D = 2 * H

NC = 2
NS = 16
NW = NC * NS
E_PER_W = N_EDGES // NW
CHUNK = 80
NCHUNK = E_PER_W // CHUNK
ROWS_PER_TILE = 624
ROWS_LAST = N_NODES - ROWS_PER_TILE * (NS - 1)
EPS = 1e-7

RBLK = 2000
NBLK = N_NODES // RBLK


def _layer_norm(h, g, b):
    mu = jnp.mean(h, axis=-1, keepdims=True)
    var = jnp.mean((h - mu) ** 2, axis=-1, keepdims=True)
    return (h - mu) * lax.rsqrt(var + 1e-5) * g + b


def _row_spec(shape):
    return pl.BlockSpec(shape, lambda i: (i,) + (0,) * (len(shape) - 1))


def _full_spec(shape):
    return pl.BlockSpec(shape, lambda i: (0,) * len(shape))


def _dense_a_body(x_ref, we_ref, be_ref, t_ref, y_ref, aux_ref):
    y = jnp.dot(x_ref[...], we_ref[...], preferred_element_type=jnp.float32)
    y = y + be_ref[...]
    y_ref[...] = y
    m = jax.nn.relu(y) + EPS
    t = t_ref[0, 0]
    mf = jnp.max(m * t, axis=0, keepdims=True)
    row = jnp.concatenate([mf, jnp.full((1, H), t, jnp.float32)], axis=1)
    aux_ref[...] = jnp.broadcast_to(row, (8, D))


def _dense_a(x, W_enc, b_enc, t):
    return pl.pallas_call(
        _dense_a_body,
        out_shape=(
            jax.ShapeDtypeStruct((N_NODES, H), jnp.float32),
            jax.ShapeDtypeStruct((8, D), jnp.float32),
        ),
    )(x, W_enc, b_enc.reshape(1, H), t.reshape(1, 1))


def _aggregate(nd_ref, x):
    nd = nd_ref[0] + nd_ref[1]
    den = nd[:, :H]
    num = nd[:, H:]
    agg = num / jnp.where(den > 0.0, den, 1.0)
    return agg + x


def _mlp(h, W1_ref, b1_ref, gm_ref, bm_ref, W2_ref, b2_ref):
    h = jnp.dot(h, W1_ref[...], preferred_element_type=jnp.float32) + b1_ref[...]
    h = _layer_norm(h, gm_ref[...], bm_ref[...])
    h = jax.nn.relu(h)
    return jnp.dot(h, W2_ref[...], preferred_element_type=jnp.float32) + b2_ref[...]


def _dense_b_body(nd_ref, y_ref, t_ref, W1_ref, b1_ref, gm_ref, bm_ref,
                  W2_ref, b2_ref, z_ref, aux_ref):
    out = _aggregate(nd_ref, y_ref[...])
    z = _mlp(out, W1_ref, b1_ref, gm_ref, bm_ref, W2_ref, b2_ref)
    z_ref[...] = z
    m = jax.nn.relu(z) + EPS
    t = t_ref[0, 0]
    mf = jnp.max(m * t, axis=0, keepdims=True)
    row = jnp.concatenate([mf, jnp.full((1, H), t, jnp.float32)], axis=1)
    aux_ref[...] = jnp.broadcast_to(row, (8, D))


def _dense_b(nd, y, t, W1, b1, g_m, b_m, W2, b2):
    return pl.pallas_call(
        _dense_b_body,
        out_shape=(
            jax.ShapeDtypeStruct((N_NODES, H), jnp.float32),
            jax.ShapeDtypeStruct((8, D), jnp.float32),
        ),
    )(nd, y, t.reshape(1, 1), W1, b1.reshape(1, D), g_m.reshape(1, D),
      b_m.reshape(1, D), W2, b2.reshape(1, H))


def _dense_c_body(nd_ref, z_ref, W1_ref, b1_ref, gm_ref, bm_ref, W2_ref,
                  b2_ref, gl_ref, bl_ref, gn_ref, bn_ref, wo_ref, bo_ref,
                  o_ref):
    out = _aggregate(nd_ref, z_ref[...])
    z2 = _mlp(out, W1_ref, b1_ref, gm_ref, bm_ref, W2_ref, b2_ref)
    h = jax.nn.relu(_layer_norm(z2, gl_ref[...], bl_ref[...]))
    cat = jnp.concatenate([z_ref[...], h], axis=1)
    cat = jax.nn.relu(_layer_norm(cat, gn_ref[...], bn_ref[...]))
    o_ref[...] = (jnp.dot(cat, wo_ref[...], preferred_element_type=jnp.float32)
                  + bo_ref[...])


def _dense_c(nd, z, W1, b1, g_m, b_m, W2, b2, g_ln1, b_ln1, g_norm, b_norm,
             W_out, b_out):
    return pl.pallas_call(
        _dense_c_body,
        grid=(NBLK,),
        in_specs=[
            pl.BlockSpec((2, RBLK, D), lambda i: (0, i, 0)),
            _row_spec((RBLK, H)),
            _full_spec((H, D)),
            _full_spec((1, D)),
            _full_spec((1, D)),
            _full_spec((1, D)),
            _full_spec((D, H)),
            _full_spec((1, H)),
            _full_spec((1, H)),
            _full_spec((1, H)),
            _full_spec((1, F_IN)),
            _full_spec((1, F_IN)),
            _full_spec((F_IN, 1)),
            _full_spec((1, 1)),
        ],
        out_specs=_row_spec((RBLK, 1)),
        out_shape=jax.ShapeDtypeStruct((N_NODES, 1), jnp.float32),
    )(nd, z, W1, b1.reshape(1, D), g_m.reshape(1, D), b_m.reshape(1, D),
      W2, b2.reshape(1, H), g_ln1.reshape(1, H), b_ln1.reshape(1, H),
      g_norm.reshape(1, F_IN), b_norm.reshape(1, F_IN), W_out,
      b_out.reshape(1, 1))


UNROLL = 4


def _sc_body(y_hbm, eidx_hbm, aux_hbm, zeros_hbm, out_hbm,
             isrc_a, isrc_b, isrc_c, idst_a, idst_b, idst_c,
             sidx_a, sidx_b, sidx_c,
             in_a, in_b, in_c, st_a, st_b, st_c, aux_v, acc,
             isem_a, isem_b, isem_c, gsem_a, gsem_b, gsem_c,
             ssem_a, ssem_b, ssem_c):
    c = lax.axis_index("c")
    s = lax.axis_index("s")
    wid = s * NC + c

    pltpu.sync_copy(aux_hbm, aux_v)

    row0 = s * ROWS_PER_TILE

    @pl.when(s < NS - 1)
    def _():
        pltpu.sync_copy(zeros_hbm.at[pl.ds(row0, ROWS_PER_TILE)],
                        acc.at[pl.ds(row0, ROWS_PER_TILE)])

    @pl.when(s == NS - 1)
    def _():
        pltpu.sync_copy(zeros_hbm.at[pl.ds(row0, ROWS_LAST)],
                        acc.at[pl.ds(row0, ROWS_LAST)])

    plsc.subcore_barrier()

    mfs = [aux_v[0, pl.ds(16 * g, 16)] for g in range(H // 16)]
    tv = aux_v[0, pl.ds(H, 16)]

    ebase = wid * E_PER_W

    def fetch_idx(j, idx_s, idx_d, sem):
        pltpu.async_copy(
            eidx_hbm.at[0].at[pl.ds(ebase + j * CHUNK, CHUNK)], idx_s, sem)
        pltpu.async_copy(
            eidx_hbm.at[1].at[pl.ds(ebase + j * CHUNK, CHUNK)], idx_d, sem)

    def wait_idx(j, idx_s, idx_d, sem):
        pltpu.make_async_copy(
            eidx_hbm.at[0].at[pl.ds(ebase + j * CHUNK, CHUNK)], idx_s, sem).wait()
        pltpu.make_async_copy(
            eidx_hbm.at[1].at[pl.ds(ebase + j * CHUNK, CHUNK)], idx_d, sem).wait()

    def gather(idx_s, inbuf, sem):
        return pltpu.async_copy(y_hbm.at[idx_s], inbuf, sem)

    def wait_gather(idx_s, inbuf, sem):
        pltpu.make_async_copy(y_hbm.at[idx_s], inbuf, sem).wait()

    def scatter(stbuf, sidx, sem):
        return pltpu.async_copy(stbuf, acc.at[sidx], sem, add=True)

    def wait_scatter(stbuf, sidx, sem):
        pltpu.make_async_copy(stbuf, acc.at[sidx], sem).wait()

    def copy_dst_idx(idx_d, sidx):
        for k in range(CHUNK // 16):
            sidx[pl.ds(16 * k, 16)] = idx_d[pl.ds(16 * k, 16)]

    def compute(inbuf, stbuf):
        NG = H // 16

        def rows(r, carry):
            base = r * UNROLL
            idxs = [(k, g) for k in range(UNROLL) for g in range(NG)]
            ys = [inbuf[base + k, pl.ds(16 * g, 16)] for k, g in idxs]
            ms = [jnp.maximum(yv, 0.0) + EPS for yv in ys]
            ps = [jnp.exp(ms[i] * tv - mfs[g]) for i, (k, g) in enumerate(idxs)]
            qs = [p * m for p, m in zip(ps, ms)]
            for i, (k, g) in enumerate(idxs):
                stbuf[base + k, pl.ds(16 * g, 16)] = ps[i]
            for i, (k, g) in enumerate(idxs):
                stbuf[base + k, pl.ds(H + 16 * g, 16)] = qs[i]
            return carry

        lax.fori_loop(0, CHUNK // UNROLL, rows, 0)

    def phase(i, j, idx_s, idx_d, sidx, inbuf, stbuf, isem, gsem, ssem):
        wait_gather(idx_s, inbuf, gsem)

        @pl.when(i > 0)
        def _():
            wait_scatter(stbuf, sidx, ssem)

        copy_dst_idx(idx_d, sidx)

        @pl.when(j + 3 < NCHUNK)
        def _():
            fetch_idx(j + 3, idx_s, idx_d, isem)

        compute(inbuf, stbuf)
        scatter(stbuf, sidx, ssem)

        @pl.when(j + 3 < NCHUNK)
        def _():
            wait_idx(j + 3, idx_s, idx_d, isem)
            gather(idx_s, inbuf, gsem)

    fetch_idx(0, isrc_a, idst_a, isem_a)
    fetch_idx(1, isrc_b, idst_b, isem_b)
    fetch_idx(2, isrc_c, idst_c, isem_c)
    wait_idx(0, isrc_a, idst_a, isem_a)
    wait_idx(1, isrc_b, idst_b, isem_b)
    wait_idx(2, isrc_c, idst_c, isem_c)
    gather(isrc_a, in_a, gsem_a)
    gather(isrc_b, in_b, gsem_b)
    gather(isrc_c, in_c, gsem_c)

    def triple(i, carry):
        j0 = 3 * i
        phase(i, j0, isrc_a, idst_a, sidx_a, in_a, st_a, isem_a, gsem_a, ssem_a)
        phase(i, j0 + 1, isrc_b, idst_b, sidx_b, in_b, st_b, isem_b, gsem_b, ssem_b)
        phase(i, j0 + 2, isrc_c, idst_c, sidx_c, in_c, st_c, isem_c, gsem_c, ssem_c)
        return carry

    lax.fori_loop(0, NCHUNK // 3, triple, 0)

    _tail = NCHUNK - 3 * (NCHUNK // 3)
    _sets = [(isrc_a, idst_a, sidx_a, in_a, st_a, isem_a, gsem_a, ssem_a),
             (isrc_b, idst_b, sidx_b, in_b, st_b, isem_b, gsem_b, ssem_b),
             (isrc_c, idst_c, sidx_c, in_c, st_c, isem_c, gsem_c, ssem_c)]
    for k in range(_tail):
        phase(1, NCHUNK - _tail + k, *_sets[k])

    wait_scatter(st_a, sidx_a, ssem_a)
    wait_scatter(st_b, sidx_b, ssem_b)
    wait_scatter(st_c, sidx_c, ssem_c)
    plsc.subcore_barrier()

    @pl.when(s < NS - 1)
    def _():
        pltpu.sync_copy(acc.at[pl.ds(row0, ROWS_PER_TILE)],
                        out_hbm.at[c].at[pl.ds(row0, ROWS_PER_TILE)])

    @pl.when(s == NS - 1)
    def _():
        pltpu.sync_copy(acc.at[pl.ds(row0, ROWS_LAST)],
                        out_hbm.at[c].at[pl.ds(row0, ROWS_LAST)])


@functools.cache
def _make_sc_scatter():
    return pl.kernel(
        _sc_body,
        out_type=jax.ShapeDtypeStruct((NC, N_NODES, D), jnp.float32),
        mesh=plsc.VectorSubcoreMesh(core_axis_name="c", subcore_axis_name="s",
                                    num_cores=NC, num_subcores=NS),
        compiler_params=pltpu.CompilerParams(use_tc_tiling_on_sc=False),
        scratch_types=(
            [pltpu.VMEM((CHUNK,), jnp.int32)] * 6
            + [pltpu.VMEM((CHUNK,), jnp.int32)] * 3
            + [pltpu.VMEM((CHUNK, H), jnp.float32)] * 3
            + [pltpu.VMEM((CHUNK, D), jnp.float32)] * 3
            + [pltpu.VMEM((8, D), jnp.float32)]
            + [pltpu.VMEM_SHARED((N_NODES, D), jnp.float32)]
            + [pltpu.SemaphoreType.DMA] * 9
        ),
    )


def _sc_scatter(y, eidx, aux, zeros):
    return _make_sc_scatter()(y, eidx, aux, zeros)


def kernel(x, edge_index, W_enc, b_enc, t, W1, b1, g_m, b_m, W2, b2,
           g_ln1, b_ln1, g_norm, b_norm, W_out, b_out):
    zeros = jnp.zeros((N_NODES, D), jnp.float32)

    y, aux1 = _dense_a(x, W_enc, b_enc, t)
    nd1 = _sc_scatter(y, edge_index, aux1, zeros)
    z, aux2 = _dense_b(nd1, y, t, W1, b1, g_m, b_m, W2, b2)
    nd2 = _sc_scatter(z, edge_index, aux2, zeros)
    return _dense_c(nd2, z, W1, b1, g_m, b_m, W2, b2, g_ln1, b_ln1,
                    g_norm, b_norm, W_out, b_out)


# speedup vs baseline: 1.1314x; 1.0165x over previous
import functools

import jax
import jax.numpy as jnp
from jax import lax
from jax.experimental import pallas as pl
from jax.experimental.pallas import tpu as pltpu
from jax.experimental.pallas import tpu_sc as plsc

N_NODES = 10000
N_EDGES = 320000
F_IN = 128
H = 64
D = 2 * H

NC = 2
NS = 16
NW = NC * NS
E_PER_W = N_EDGES // NW
CHUNK = 80
NCHUNK = E_PER_W // CHUNK
ROWS_PER_TILE = 624
ROWS_LAST = N_NODES - ROWS_PER_TILE * (NS - 1)
EPS = 1e-7

RBLK = 2000
NBLK = N_NODES // RBLK


def _layer_norm(h, g, b):
    mu = jnp.mean(h, axis=-1, keepdims=True)
    var = jnp.mean((h - mu) ** 2, axis=-1, keepdims=True)
    return (h - mu) * lax.rsqrt(var + 1e-5) * g + b


def _row_spec(shape):
    return pl.BlockSpec(shape, lambda i: (i,) + (0,) * (len(shape) - 1))


def _full_spec(shape):
    return pl.BlockSpec(shape, lambda i: (0,) * len(shape))


def _dense_a_body(x_ref, we_ref, be_ref, t_ref, y_ref, aux_ref):
    y = jnp.dot(x_ref[...], we_ref[...], preferred_element_type=jnp.float32)
    y = y + be_ref[...]
    y_ref[...] = y
    m = jax.nn.relu(y) + EPS
    t = t_ref[0, 0]
    mf = jnp.max(m * t, axis=0, keepdims=True)
    row = jnp.concatenate([mf, jnp.full((1, H), t, jnp.float32)], axis=1)
    aux_ref[...] = jnp.broadcast_to(row, (8, D))


def _dense_a(x, W_enc, b_enc, t):
    return pl.pallas_call(
        _dense_a_body,
        out_shape=(
            jax.ShapeDtypeStruct((N_NODES, H), jnp.float32),
            jax.ShapeDtypeStruct((8, D), jnp.float32),
        ),
    )(x, W_enc, b_enc.reshape(1, H), t.reshape(1, 1))


def _aggregate(nd_ref, x):
    nd = nd_ref[0] + nd_ref[1]
    den = nd[:, :H]
    num = nd[:, H:]
    agg = num / jnp.where(den > 0.0, den, 1.0)
    return agg + x


def _mlp(h, W1_ref, b1_ref, gm_ref, bm_ref, W2_ref, b2_ref):
    h = jnp.dot(h, W1_ref[...], preferred_element_type=jnp.float32) + b1_ref[...]
    h = _layer_norm(h, gm_ref[...], bm_ref[...])
    h = jax.nn.relu(h)
    return jnp.dot(h, W2_ref[...], preferred_element_type=jnp.float32) + b2_ref[...]


def _dense_b_body(nd_ref, y_ref, t_ref, W1_ref, b1_ref, gm_ref, bm_ref,
                  W2_ref, b2_ref, z_ref, aux_ref):
    out = _aggregate(nd_ref, y_ref[...])
    z = _mlp(out, W1_ref, b1_ref, gm_ref, bm_ref, W2_ref, b2_ref)
    z_ref[...] = z
    m = jax.nn.relu(z) + EPS
    t = t_ref[0, 0]
    mf = jnp.max(m * t, axis=0, keepdims=True)
    row = jnp.concatenate([mf, jnp.full((1, H), t, jnp.float32)], axis=1)
    aux_ref[...] = jnp.broadcast_to(row, (8, D))


def _dense_b(nd, y, t, W1, b1, g_m, b_m, W2, b2):
    return pl.pallas_call(
        _dense_b_body,
        out_shape=(
            jax.ShapeDtypeStruct((N_NODES, H), jnp.float32),
            jax.ShapeDtypeStruct((8, D), jnp.float32),
        ),
    )(nd, y, t.reshape(1, 1), W1, b1.reshape(1, D), g_m.reshape(1, D),
      b_m.reshape(1, D), W2, b2.reshape(1, H))


def _dense_c_body(nd_ref, z_ref, W1_ref, b1_ref, gm_ref, bm_ref, W2_ref,
                  b2_ref, gl_ref, bl_ref, gn_ref, bn_ref, wo_ref, bo_ref,
                  o_ref):
    out = _aggregate(nd_ref, z_ref[...])
    z2 = _mlp(out, W1_ref, b1_ref, gm_ref, bm_ref, W2_ref, b2_ref)
    h = jax.nn.relu(_layer_norm(z2, gl_ref[...], bl_ref[...]))
    cat = jnp.concatenate([z_ref[...], h], axis=1)
    cat = jax.nn.relu(_layer_norm(cat, gn_ref[...], bn_ref[...]))
    o_ref[...] = (jnp.dot(cat, wo_ref[...], preferred_element_type=jnp.float32)
                  + bo_ref[...])


def _dense_c(nd, z, W1, b1, g_m, b_m, W2, b2, g_ln1, b_ln1, g_norm, b_norm,
             W_out, b_out):
    return pl.pallas_call(
        _dense_c_body,
        grid=(NBLK,),
        in_specs=[
            pl.BlockSpec((2, RBLK, D), lambda i: (0, i, 0)),
            _row_spec((RBLK, H)),
            _full_spec((H, D)),
            _full_spec((1, D)),
            _full_spec((1, D)),
            _full_spec((1, D)),
            _full_spec((D, H)),
            _full_spec((1, H)),
            _full_spec((1, H)),
            _full_spec((1, H)),
            _full_spec((1, F_IN)),
            _full_spec((1, F_IN)),
            _full_spec((F_IN, 1)),
            _full_spec((1, 1)),
        ],
        out_specs=_row_spec((RBLK, 1)),
        out_shape=jax.ShapeDtypeStruct((N_NODES, 1), jnp.float32),
    )(nd, z, W1, b1.reshape(1, D), g_m.reshape(1, D), b_m.reshape(1, D),
      W2, b2.reshape(1, H), g_ln1.reshape(1, H), b_ln1.reshape(1, H),
      g_norm.reshape(1, F_IN), b_norm.reshape(1, F_IN), W_out,
      b_out.reshape(1, 1))


UNROLL = 4


def _sc_body(y_hbm, eidx_hbm, aux_hbm, zeros_hbm, out_hbm,
             isrc_a, isrc_b, isrc_c, idst_a, idst_b, idst_c,
             sidx_a, sidx_b, sidx_c,
             in_a, in_b, in_c, st_a, st_b, st_c, aux_v, acc,
             isem_a, isem_b, isem_c, gsem_a, gsem_b, gsem_c,
             ssem_a, ssem_b, ssem_c, zsem):
    c = lax.axis_index("c")
    s = lax.axis_index("s")
    wid = s * NC + c

    pltpu.sync_copy(aux_hbm, aux_v)

    row0 = s * ROWS_PER_TILE

    @pl.when(s < NS - 1)
    def _():
        pltpu.async_copy(zeros_hbm.at[pl.ds(row0, ROWS_PER_TILE)],
                         acc.at[pl.ds(row0, ROWS_PER_TILE)], zsem)

    @pl.when(s == NS - 1)
    def _():
        pltpu.async_copy(zeros_hbm.at[pl.ds(row0, ROWS_LAST)],
                         acc.at[pl.ds(row0, ROWS_LAST)], zsem)

    mfs = [aux_v[0, pl.ds(16 * g, 16)] for g in range(H // 16)]
    tv = aux_v[0, pl.ds(H, 16)]

    ebase = wid * E_PER_W

    def fetch_idx(j, idx_s, idx_d, sem):
        pltpu.async_copy(
            eidx_hbm.at[0].at[pl.ds(ebase + j * CHUNK, CHUNK)], idx_s, sem)
        pltpu.async_copy(
            eidx_hbm.at[1].at[pl.ds(ebase + j * CHUNK, CHUNK)], idx_d, sem)

    def wait_idx(j, idx_s, idx_d, sem):
        pltpu.make_async_copy(
            eidx_hbm.at[0].at[pl.ds(ebase + j * CHUNK, CHUNK)], idx_s, sem).wait()
        pltpu.make_async_copy(
            eidx_hbm.at[1].at[pl.ds(ebase + j * CHUNK, CHUNK)], idx_d, sem).wait()

    def gather(idx_s, inbuf, sem):
        return pltpu.async_copy(y_hbm.at[idx_s], inbuf, sem)

    def wait_gather(idx_s, inbuf, sem):
        pltpu.make_async_copy(y_hbm.at[idx_s], inbuf, sem).wait()

    def scatter(stbuf, sidx, sem):
        return pltpu.async_copy(stbuf, acc.at[sidx], sem, add=True)

    def wait_scatter(stbuf, sidx, sem):
        pltpu.make_async_copy(stbuf, acc.at[sidx], sem).wait()

    def copy_dst_idx(idx_d, sidx):
        for k in range(CHUNK // 16):
            sidx[pl.ds(16 * k, 16)] = idx_d[pl.ds(16 * k, 16)]

    def compute(inbuf, stbuf):
        NG = H // 16

        def rows(r, carry):
            base = r * UNROLL
            idxs = [(k, g) for k in range(UNROLL) for g in range(NG)]
            ys = [inbuf[base + k, pl.ds(16 * g, 16)] for k, g in idxs]
            ms = [jnp.maximum(yv, 0.0) + EPS for yv in ys]
            ps = [jnp.exp(ms[i] * tv - mfs[g]) for i, (k, g) in enumerate(idxs)]
            qs = [p * m for p, m in zip(ps, ms)]
            for i, (k, g) in enumerate(idxs):
                stbuf[base + k, pl.ds(16 * g, 16)] = ps[i]
            for i, (k, g) in enumerate(idxs):
                stbuf[base + k, pl.ds(H + 16 * g, 16)] = qs[i]
            return carry

        lax.fori_loop(0, CHUNK // UNROLL, rows, 0)

    def phase(i, j, idx_s, idx_d, sidx, inbuf, stbuf, isem, gsem, ssem):
        wait_gather(idx_s, inbuf, gsem)

        @pl.when(i > 0)
        def _():
            wait_scatter(stbuf, sidx, ssem)

        copy_dst_idx(idx_d, sidx)

        @pl.when(j + 3 < NCHUNK)
        def _():
            fetch_idx(j + 3, idx_s, idx_d, isem)

        compute(inbuf, stbuf)
        scatter(stbuf, sidx, ssem)

        @pl.when(j + 3 < NCHUNK)
        def _():
            wait_idx(j + 3, idx_s, idx_d, isem)
            gather(idx_s, inbuf, gsem)

    fetch_idx(0, isrc_a, idst_a, isem_a)
    fetch_idx(1, isrc_b, idst_b, isem_b)
    fetch_idx(2, isrc_c, idst_c, isem_c)
    wait_idx(0, isrc_a, idst_a, isem_a)
    wait_idx(1, isrc_b, idst_b, isem_b)
    wait_idx(2, isrc_c, idst_c, isem_c)
    gather(isrc_a, in_a, gsem_a)
    gather(isrc_b, in_b, gsem_b)
    gather(isrc_c, in_c, gsem_c)

    @pl.when(s < NS - 1)
    def _():
        pltpu.make_async_copy(zeros_hbm.at[pl.ds(row0, ROWS_PER_TILE)],
                              acc.at[pl.ds(row0, ROWS_PER_TILE)], zsem).wait()

    @pl.when(s == NS - 1)
    def _():
        pltpu.make_async_copy(zeros_hbm.at[pl.ds(row0, ROWS_LAST)],
                              acc.at[pl.ds(row0, ROWS_LAST)], zsem).wait()

    plsc.subcore_barrier()

    def triple(i, carry):
        j0 = 3 * i
        phase(i, j0, isrc_a, idst_a, sidx_a, in_a, st_a, isem_a, gsem_a, ssem_a)
        phase(i, j0 + 1, isrc_b, idst_b, sidx_b, in_b, st_b, isem_b, gsem_b, ssem_b)
        phase(i, j0 + 2, isrc_c, idst_c, sidx_c, in_c, st_c, isem_c, gsem_c, ssem_c)
        return carry

    lax.fori_loop(0, NCHUNK // 3, triple, 0)

    _tail = NCHUNK - 3 * (NCHUNK // 3)
    _sets = [(isrc_a, idst_a, sidx_a, in_a, st_a, isem_a, gsem_a, ssem_a),
             (isrc_b, idst_b, sidx_b, in_b, st_b, isem_b, gsem_b, ssem_b),
             (isrc_c, idst_c, sidx_c, in_c, st_c, isem_c, gsem_c, ssem_c)]
    for k in range(_tail):
        phase(1, NCHUNK - _tail + k, *_sets[k])

    wait_scatter(st_a, sidx_a, ssem_a)
    wait_scatter(st_b, sidx_b, ssem_b)
    wait_scatter(st_c, sidx_c, ssem_c)
    plsc.subcore_barrier()

    @pl.when(s < NS - 1)
    def _():
        pltpu.sync_copy(acc.at[pl.ds(row0, ROWS_PER_TILE)],
                        out_hbm.at[c].at[pl.ds(row0, ROWS_PER_TILE)])

    @pl.when(s == NS - 1)
    def _():
        pltpu.sync_copy(acc.at[pl.ds(row0, ROWS_LAST)],
                        out_hbm.at[c].at[pl.ds(row0, ROWS_LAST)])


@functools.cache
def _make_sc_scatter():
    return pl.kernel(
        _sc_body,
        out_type=jax.ShapeDtypeStruct((NC, N_NODES, D), jnp.float32),
        mesh=plsc.VectorSubcoreMesh(core_axis_name="c", subcore_axis_name="s",
                                    num_cores=NC, num_subcores=NS),
        compiler_params=pltpu.CompilerParams(use_tc_tiling_on_sc=False),
        scratch_types=(
            [pltpu.VMEM((CHUNK,), jnp.int32)] * 6
            + [pltpu.VMEM((CHUNK,), jnp.int32)] * 3
            + [pltpu.VMEM((CHUNK, H), jnp.float32)] * 3
            + [pltpu.VMEM((CHUNK, D), jnp.float32)] * 3
            + [pltpu.VMEM((8, D), jnp.float32)]
            + [pltpu.VMEM_SHARED((N_NODES, D), jnp.float32)]
            + [pltpu.SemaphoreType.DMA] * 10
        ),
    )


def _sc_scatter(y, eidx, aux, zeros):
    return _make_sc_scatter()(y, eidx, aux, zeros)


def kernel(x, edge_index, W_enc, b_enc, t, W1, b1, g_m, b_m, W2, b2,
           g_ln1, b_ln1, g_norm, b_norm, W_out, b_out):
    zeros = jnp.zeros((N_NODES, D), jnp.float32)

    y, aux1 = _dense_a(x, W_enc, b_enc, t)
    nd1 = _sc_scatter(y, edge_index, aux1, zeros)
    z, aux2 = _dense_b(nd1, y, t, W1, b1, g_m, b_m, W2, b2)
    nd2 = _sc_scatter(z, edge_index, aux2, zeros)
    return _dense_c(nd2, z, W1, b1, g_m, b_m, W2, b2, g_ln1, b_ln1,
                    g_norm, b_norm, W_out, b_out)
